# baseline (speedup 1.0000x reference)
import functools

import jax
import jax.numpy as jnp
from jax import lax
from jax.experimental import pallas as pl
from jax.experimental.pallas import tpu as pltpu
from jax.experimental.pallas import tpu_sc as plsc

B = 1024
D = 256
N = 16384
IMG = 4096
TILE = 2048
NT = N // TILE

NC, NS = 2, 16
NW = NC * NS
B_PER_W = B // NW
CHUNK = 16
N_CHUNKS = B_PER_W // CHUNK


def _topk_body(q_ref, f_ref, scores_ref, idx_ref, qn_ref):
    i = pl.program_id(0)

    @pl.when(i == 0)
    def _():
        q = q_ref[...]
        n = jnp.sqrt(jnp.sum(q * q, axis=1, keepdims=True))
        qn_ref[...] = q / jnp.clip(n, 1e-12, None)

    sim = lax.dot_general(
        qn_ref[...], f_ref[...],
        dimension_numbers=(((1,), (1,)), ((), ())),
        preferred_element_type=jnp.float32,
    )
    m = jnp.max(sim, axis=1, keepdims=True)
    pos = lax.broadcasted_iota(jnp.int32, (B, TILE), 1)
    a = jnp.min(jnp.where(sim == m, pos, TILE), axis=1, keepdims=True) + i * TILE

    @pl.when(i == 0)
    def _():
        scores_ref[...] = m
        idx_ref[...] = a

    @pl.when(i > 0)
    def _():
        prev = scores_ref[...]
        better = m > prev
        scores_ref[...] = jnp.where(better, m, prev)
        idx_ref[...] = jnp.where(better, a, idx_ref[...])


def _topk(q, features):
    return pl.pallas_call(
        _topk_body,
        grid=(NT,),
        in_specs=[
            pl.BlockSpec((B, D), lambda i: (0, 0)),
            pl.BlockSpec((TILE, D), lambda i: (i, 0)),
        ],
        out_specs=[
            pl.BlockSpec((B, 1), lambda i: (0, 0)),
            pl.BlockSpec((B, 1), lambda i: (0, 0)),
        ],
        out_shape=[
            jax.ShapeDtypeStruct((B, 1), jnp.float32),
            jax.ShapeDtypeStruct((B, 1), jnp.int32),
        ],
        scratch_shapes=[pltpu.VMEM((B, D), jnp.float32)],
    )(q, features)


TK = 2048
MT = 512
NKT = N // TK
NMT = IMG // MT
SELT = 2048


def _sel_body(idx_ref, sel_ref):
    i = pl.program_id(0)
    kio = lax.broadcasted_iota(jnp.int32, (SELT, B), 0) + i * SELT
    sel_ref[...] = (kio == idx_ref[...]).astype(jnp.bfloat16)


def _build_sel(idx2):
    return pl.pallas_call(
        _sel_body,
        grid=(N // SELT,),
        in_specs=[pl.BlockSpec((1, B), lambda i: (0, 0))],
        out_specs=pl.BlockSpec((SELT, B), lambda i: (i, 0)),
        out_shape=jax.ShapeDtypeStruct((N, B), jnp.bfloat16),
    )(idx2)


def _gmm_body(imgT_ref, sel_ref, out_ref):
    k = pl.program_id(1)
    acc = lax.dot_general(
        imgT_ref[...].astype(jnp.bfloat16), sel_ref[...],
        dimension_numbers=(((1,), (0,)), ((), ())),
        preferred_element_type=jnp.float32,
    )

    @pl.when(k == 0)
    def _():
        out_ref[...] = acc

    @pl.when(k > 0)
    def _():
        out_ref[...] += acc


def _gather_mm(imgT, sel):
    return pl.pallas_call(
        _gmm_body,
        grid=(NMT, NKT),
        in_specs=[
            pl.BlockSpec((MT, TK), lambda m, k: (m, k)),
            pl.BlockSpec((TK, B), lambda m, k: (k, 0)),
        ],
        out_specs=pl.BlockSpec((MT, B), lambda m, k: (m, 0)),
        out_shape=jax.ShapeDtypeStruct((IMG, B), jnp.float32),
    )(imgT, sel)


def kernel(query_features, features, images):
    scores2, idx2 = _topk(query_features, features)
    imgT = images.transpose(1, 2, 3, 0).reshape(IMG, N)
    sel = _build_sel(idx2.reshape(1, B))
    outT = _gather_mm(imgT, sel)
    out = outT.reshape(1, 64, 64, B).transpose(3, 0, 1, 2)
    return out, scores2.reshape(B)


# speedup vs baseline: 1.0882x; 1.0882x over previous
import functools

import jax
import jax.numpy as jnp
from jax import lax
from jax.experimental import pallas as pl
from jax.experimental.pallas import tpu as pltpu
from jax.experimental.pallas import tpu_sc as plsc

B = 1024
D = 256
N = 16384
IMG = 4096
TILE = 2048
NT = N // TILE

NC, NS = 2, 16
NW = NC * NS
B_PER_W = B // NW
CHUNK = 16
N_CHUNKS = B_PER_W // CHUNK


def _topk_body(q_ref, f_ref, scores_ref, idx_ref, qn_ref):
    i = pl.program_id(0)

    @pl.when(i == 0)
    def _():
        q = q_ref[...]
        n = jnp.sqrt(jnp.sum(q * q, axis=1, keepdims=True))
        qn_ref[...] = q / jnp.clip(n, 1e-12, None)

    sim = lax.dot_general(
        qn_ref[...], f_ref[...],
        dimension_numbers=(((1,), (1,)), ((), ())),
        preferred_element_type=jnp.float32,
    )
    m = jnp.max(sim, axis=1, keepdims=True)
    pos = lax.broadcasted_iota(jnp.int32, (B, TILE), 1)
    a = jnp.min(jnp.where(sim == m, pos, TILE), axis=1, keepdims=True) + i * TILE

    @pl.when(i == 0)
    def _():
        scores_ref[...] = m
        idx_ref[...] = a

    @pl.when(i > 0)
    def _():
        prev = scores_ref[...]
        better = m > prev
        scores_ref[...] = jnp.where(better, m, prev)
        idx_ref[...] = jnp.where(better, a, idx_ref[...])


def _topk(q, features):
    return pl.pallas_call(
        _topk_body,
        grid=(NT,),
        in_specs=[
            pl.BlockSpec((B, D), lambda i: (0, 0)),
            pl.BlockSpec((TILE, D), lambda i: (i, 0)),
        ],
        out_specs=[
            pl.BlockSpec((B, 1), lambda i: (0, 0)),
            pl.BlockSpec((B, 1), lambda i: (0, 0)),
        ],
        out_shape=[
            jax.ShapeDtypeStruct((B, 1), jnp.float32),
            jax.ShapeDtypeStruct((B, 1), jnp.int32),
        ],
        scratch_shapes=[pltpu.VMEM((B, D), jnp.float32)],
    )(q, features)


TK = 2048
MT = 512
NKT = N // TK
NMT = IMG // MT
SELT = 2048


def _sel_body(idx_ref, sel_ref):
    i = pl.program_id(0)
    kio = lax.broadcasted_iota(jnp.int32, (SELT, B), 0) + i * SELT
    sel_ref[...] = (kio == idx_ref[...]).astype(jnp.bfloat16)


def _build_sel(idx2):
    return pl.pallas_call(
        _sel_body,
        grid=(N // SELT,),
        in_specs=[pl.BlockSpec((1, B), lambda i: (0, 0))],
        out_specs=pl.BlockSpec((SELT, B), lambda i: (i, 0)),
        out_shape=jax.ShapeDtypeStruct((N, B), jnp.bfloat16),
    )(idx2)


def _gmm_body(imgT_ref, sel_ref, out_ref):
    k = pl.program_id(1)
    sblk = sel_ref[pl.ds(k * TK, TK), :]
    acc = lax.dot_general(
        imgT_ref[...].astype(jnp.bfloat16), sblk,
        dimension_numbers=(((1,), (0,)), ((), ())),
        preferred_element_type=jnp.float32,
    )

    @pl.when(k == 0)
    def _():
        out_ref[...] = acc

    @pl.when(k > 0)
    def _():
        out_ref[...] += acc


def _gather_mm(imgT, sel):
    return pl.pallas_call(
        _gmm_body,
        grid=(NMT, NKT),
        in_specs=[
            pl.BlockSpec((MT, TK), lambda m, k: (m, k)),
            pl.BlockSpec((N, B), lambda m, k: (0, 0)),
        ],
        out_specs=pl.BlockSpec((MT, B), lambda m, k: (m, 0)),
        out_shape=jax.ShapeDtypeStruct((IMG, B), jnp.float32),
    )(imgT, sel)


def kernel(query_features, features, images):
    scores2, idx2 = _topk(query_features, features)
    imgT = images.transpose(1, 2, 3, 0).reshape(IMG, N)
    sel = _build_sel(idx2.reshape(1, B))
    outT = _gather_mm(imgT, sel)
    out = outT.reshape(1, 64, 64, B).transpose(3, 0, 1, 2)
    return out, scores2.reshape(B)


# speedup vs baseline: 1.1691x; 1.0743x over previous
import functools

import jax
import jax.numpy as jnp
from jax import lax
from jax.experimental import pallas as pl
from jax.experimental.pallas import tpu as pltpu
from jax.experimental.pallas import tpu_sc as plsc

B = 1024
D = 256
N = 16384
IMG = 4096
TILE = 2048
NT = N // TILE

NC, NS = 2, 16
NW = NC * NS
B_PER_W = B // NW
CHUNK = 16
N_CHUNKS = B_PER_W // CHUNK


def _topk_body(q_ref, f_ref, scores_ref, idx_ref, qn_ref):
    i = pl.program_id(0)

    @pl.when(i == 0)
    def _():
        q = q_ref[...]
        n = jnp.sqrt(jnp.sum(q * q, axis=1, keepdims=True))
        qn_ref[...] = q / jnp.clip(n, 1e-12, None)

    sim = lax.dot_general(
        qn_ref[...], f_ref[...],
        dimension_numbers=(((1,), (1,)), ((), ())),
        preferred_element_type=jnp.float32,
    )
    m = jnp.max(sim, axis=1, keepdims=True)
    pos = lax.broadcasted_iota(jnp.int32, (B, TILE), 1)
    a = jnp.min(jnp.where(sim == m, pos, TILE), axis=1, keepdims=True) + i * TILE

    @pl.when(i == 0)
    def _():
        scores_ref[...] = m
        idx_ref[...] = a

    @pl.when(i > 0)
    def _():
        prev = scores_ref[...]
        better = m > prev
        scores_ref[...] = jnp.where(better, m, prev)
        idx_ref[...] = jnp.where(better, a, idx_ref[...])


def _topk(q, features):
    return pl.pallas_call(
        _topk_body,
        grid=(NT,),
        in_specs=[
            pl.BlockSpec((B, D), lambda i: (0, 0)),
            pl.BlockSpec((TILE, D), lambda i: (i, 0)),
        ],
        out_specs=[
            pl.BlockSpec((B, 1), lambda i: (0, 0)),
            pl.BlockSpec((B, 1), lambda i: (0, 0)),
        ],
        out_shape=[
            jax.ShapeDtypeStruct((B, 1), jnp.float32),
            jax.ShapeDtypeStruct((B, 1), jnp.int32),
        ],
        scratch_shapes=[pltpu.VMEM((B, D), jnp.float32)],
    )(q, features)


TK = 4096
MT = 512
NKT = N // TK
NMT = IMG // MT
SELT = 2048


def _sel_body(idx_ref, sel_ref):
    i = pl.program_id(0)
    kio = lax.broadcasted_iota(jnp.int32, (SELT, B), 0) + i * SELT
    sel_ref[...] = (kio == idx_ref[...]).astype(jnp.bfloat16)


def _build_sel(idx2):
    return pl.pallas_call(
        _sel_body,
        grid=(N // SELT,),
        in_specs=[pl.BlockSpec((1, B), lambda i: (0, 0))],
        out_specs=pl.BlockSpec((SELT, B), lambda i: (i, 0)),
        out_shape=jax.ShapeDtypeStruct((N, B), jnp.bfloat16),
    )(idx2)


def _gmm_body(imgT_ref, sel_ref, out_ref):
    k = pl.program_id(1)
    sblk = sel_ref[pl.ds(k * TK, TK), :]
    acc = lax.dot_general(
        imgT_ref[...].astype(jnp.bfloat16), sblk,
        dimension_numbers=(((1,), (0,)), ((), ())),
        preferred_element_type=jnp.float32,
    )

    @pl.when(k == 0)
    def _():
        out_ref[...] = acc

    @pl.when(k > 0)
    def _():
        out_ref[...] += acc


def _gather_mm(imgT, sel):
    return pl.pallas_call(
        _gmm_body,
        grid=(NMT, NKT),
        in_specs=[
            pl.BlockSpec((MT, TK), lambda m, k: (m, k)),
            pl.BlockSpec((N, B), lambda m, k: (0, 0)),
        ],
        out_specs=pl.BlockSpec((MT, B), lambda m, k: (m, 0)),
        out_shape=jax.ShapeDtypeStruct((IMG, B), jnp.float32),
    )(imgT, sel)


def kernel(query_features, features, images):
    scores2, idx2 = _topk(query_features, features)
    imgT = images.transpose(1, 2, 3, 0).reshape(IMG, N)
    sel = _build_sel(idx2.reshape(1, B))
    outT = _gather_mm(imgT, sel)
    out = outT.reshape(1, 64, 64, B).transpose(3, 0, 1, 2)
    return out, scores2.reshape(B)


# speedup vs baseline: 1.3112x; 1.1215x over previous
import functools

import jax
import jax.numpy as jnp
from jax import lax
from jax.experimental import pallas as pl
from jax.experimental.pallas import tpu as pltpu
from jax.experimental.pallas import tpu_sc as plsc

B = 1024
D = 256
N = 16384
IMG = 4096
TILE = 2048
NT = N // TILE

NC, NS = 2, 16
NW = NC * NS
B_PER_W = B // NW
CHUNK = 16
N_CHUNKS = B_PER_W // CHUNK


def _topk_body(q_ref, f_ref, scores_ref, idx_ref, qn_ref):
    i = pl.program_id(0)

    @pl.when(i == 0)
    def _():
        q = q_ref[...]
        n = jnp.sqrt(jnp.sum(q * q, axis=1, keepdims=True))
        qn_ref[...] = q / jnp.clip(n, 1e-12, None)

    sim = lax.dot_general(
        qn_ref[...], f_ref[...],
        dimension_numbers=(((1,), (1,)), ((), ())),
        preferred_element_type=jnp.float32,
    )
    m = jnp.max(sim, axis=1, keepdims=True)
    pos = lax.broadcasted_iota(jnp.int32, (B, TILE), 1)
    a = jnp.min(jnp.where(sim == m, pos, TILE), axis=1, keepdims=True) + i * TILE

    @pl.when(i == 0)
    def _():
        scores_ref[...] = m
        idx_ref[...] = a

    @pl.when(i > 0)
    def _():
        prev = scores_ref[...]
        better = m > prev
        scores_ref[...] = jnp.where(better, m, prev)
        idx_ref[...] = jnp.where(better, a, idx_ref[...])


def _topk(q, features):
    return pl.pallas_call(
        _topk_body,
        grid=(NT,),
        in_specs=[
            pl.BlockSpec((B, D), lambda i: (0, 0)),
            pl.BlockSpec((TILE, D), lambda i: (i, 0)),
        ],
        out_specs=[
            pl.BlockSpec((B, 1), lambda i: (0, 0)),
            pl.BlockSpec((B, 1), lambda i: (0, 0)),
        ],
        out_shape=[
            jax.ShapeDtypeStruct((B, 1), jnp.float32),
            jax.ShapeDtypeStruct((B, 1), jnp.int32),
        ],
        scratch_shapes=[pltpu.VMEM((B, D), jnp.float32)],
    )(q, features)


TK = 4096
MT = 512
NKT = N // TK
NMT = IMG // MT
def _gmm_body(idx_ref, imgT_ref, out_ref):
    k = pl.program_id(1)
    kio = lax.broadcasted_iota(jnp.int32, (TK, B), 0) + k * TK
    sel = (kio == idx_ref[...]).astype(jnp.bfloat16)
    acc = lax.dot_general(
        imgT_ref[...].astype(jnp.bfloat16), sel,
        dimension_numbers=(((1,), (0,)), ((), ())),
        preferred_element_type=jnp.float32,
    )

    @pl.when(k == 0)
    def _():
        out_ref[...] = acc

    @pl.when(k > 0)
    def _():
        out_ref[...] += acc


def _gather_mm(imgT, idx2):
    return pl.pallas_call(
        _gmm_body,
        grid=(NMT, NKT),
        in_specs=[
            pl.BlockSpec((1, B), lambda m, k: (0, 0)),
            pl.BlockSpec((MT, TK), lambda m, k: (m, k)),
        ],
        out_specs=pl.BlockSpec((MT, B), lambda m, k: (m, 0)),
        out_shape=jax.ShapeDtypeStruct((IMG, B), jnp.float32),
    )(idx2, imgT)


def kernel(query_features, features, images):
    scores2, idx2 = _topk(query_features, features)
    imgT = images.transpose(1, 2, 3, 0).reshape(IMG, N)
    outT = _gather_mm(imgT, idx2.reshape(1, B))
    out = outT.reshape(1, 64, 64, B).transpose(3, 0, 1, 2)
    return out, scores2.reshape(B)
